# Initial kernel scaffold; baseline (speedup 1.0000x reference)
import jax
import jax.numpy as jnp
from jax.experimental import pallas as pl


def kernel(int_batch, table):
    raise NotImplementedError("write your pallas kernel here")


# speedup vs baseline: 1.3585x; 1.3585x over previous
import functools

import jax
import jax.numpy as jnp
from jax import lax
from jax.experimental import pallas as pl
from jax.experimental.pallas import tpu as pltpu
from jax.experimental.pallas import tpu_sc as plsc

VOCAB = 128
EMBED_DIM = 64
BATCH = 16384
SEQ = 200

NC = 2
NS = 16
LANES = 16
NW = NC * NS

N = BATCH * SEQ
PER_W = N // NW
CHUNK = 512
N_CHUNKS = PER_W // CHUNK
GROUPS = CHUNK // LANES


@functools.partial(
    pl.kernel,
    out_type=jax.ShapeDtypeStruct((N * EMBED_DIM,), jnp.float32),
    mesh=plsc.VectorSubcoreMesh(core_axis_name="c", subcore_axis_name="s"),
    scratch_types=[
        pltpu.VMEM((VOCAB * EMBED_DIM,), jnp.float32),
        pltpu.VMEM((CHUNK,), jnp.int32),
        pltpu.VMEM((CHUNK * EMBED_DIM,), jnp.float32),
    ],
    compiler_params=pltpu.CompilerParams(needs_layout_passes=False),
)
def _sc_embed(table_hbm, idx_hbm, out_hbm, table_v, idx_v, out_v):
    wid = lax.axis_index("s") * NC + lax.axis_index("c")
    base = wid * PER_W
    pltpu.sync_copy(table_hbm, table_v)
    lane = lax.iota(jnp.int32, 16)
    lane_off = lane * EMBED_DIM

    def chunk_body(ci, carry):
        off = base + ci * CHUNK
        pltpu.sync_copy(idx_hbm.at[pl.ds(off, CHUNK)], idx_v)

        def group_body(g, c2):
            iv = idx_v[pl.ds(g * LANES, LANES)]
            src = iv * EMBED_DIM
            dst = g * (LANES * EMBED_DIM) + lane_off
            for c in range(EMBED_DIM):
                vals = plsc.load_gather(table_v, [src + c])
                plsc.store_scatter(out_v, [dst + c], vals)
            return c2

        lax.fori_loop(0, GROUPS, group_body, 0)
        pltpu.sync_copy(out_v, out_hbm.at[pl.ds(off * EMBED_DIM,
                                                CHUNK * EMBED_DIM)])
        return carry

    lax.fori_loop(0, N_CHUNKS, chunk_body, 0)


def kernel(int_batch, table):
    idx_flat = int_batch.reshape(-1).astype(jnp.int32)
    table_flat = table.reshape(-1)
    out = _sc_embed(table_flat, idx_flat)
    return out.reshape(BATCH, SEQ, EMBED_DIM)
